# baseline (speedup 1.0000x reference)
import jax
import jax.numpy as jnp
from jax import lax
from jax.experimental import pallas as pl
from jax.experimental.pallas import tpu as pltpu
from jax.experimental.pallas import tpu_sc as plsc

INPUT_DIM = 1000000
PROJ_DIM = 32
BATCH = 16384

_info = plsc.get_sparse_core_info()
_NC, _NS, _L = _info.num_cores, _info.num_subcores, _info.num_lanes
_NW = _NC * _NS
_BPW = BATCH // _NW


def _sc_body(xf_hbm, ww_hbm, wc_hbm, out_hbm,
             xf_v, idx0_v, idx1_v, rows_w, rows_c, out_v, sem0, sem1):
    wid = lax.axis_index("s") * _NC + lax.axis_index("c")
    base = wid * _BPW

    pltpu.sync_copy(xf_hbm.at[pl.ds(base * 2, _BPW * 2)], xf_v)

    lane_iota = lax.iota(jnp.int32, _L)

    for k in range(_BPW // _L):
        pos = lane_iota * 2 + (k * 2 * _L)
        idx0_v[pl.ds(k * _L, _L)] = plsc.load_gather(xf_v, [pos])
        idx1_v[pl.ds(k * _L, _L)] = plsc.load_gather(xf_v, [pos + 1])

    cp0 = pltpu.async_copy(ww_hbm.at[idx0_v], rows_w, sem0)
    cp1 = pltpu.async_copy(wc_hbm.at[idx1_v], rows_c, sem1)
    cp0.wait()
    cp1.wait()

    def block_body(b, _):
        row_idx = lane_iota + b * _L
        acc = jnp.zeros((_L,), jnp.float32)
        for j in range(PROJ_DIM):
            col = jnp.full((_L,), j, jnp.int32)
            gw = plsc.load_gather(rows_w, [row_idx, col])
            gc = plsc.load_gather(rows_c, [row_idx, col])
            acc = acc + gw * gc
        out_v[pl.ds(b * _L, _L)] = 1.0 / (1.0 + jnp.exp(-acc))
        return 0

    lax.fori_loop(0, _BPW // _L, block_body, 0)

    pltpu.sync_copy(out_v, out_hbm.at[pl.ds(base, _BPW)])


@jax.jit
def _run(xf, W_w, W_c):
    mesh = plsc.VectorSubcoreMesh(core_axis_name="c", subcore_axis_name="s")
    f = pl.kernel(
        _sc_body,
        mesh=mesh,
        compiler_params=pltpu.CompilerParams(
            use_tc_tiling_on_sc=False, needs_layout_passes=False),
        out_type=jax.ShapeDtypeStruct((BATCH,), jnp.float32),
        scratch_types=[
            pltpu.VMEM((2 * _BPW,), jnp.int32),
            pltpu.VMEM((_BPW,), jnp.int32),
            pltpu.VMEM((_BPW,), jnp.int32),
            pltpu.VMEM((_BPW, PROJ_DIM), jnp.float32),
            pltpu.VMEM((_BPW, PROJ_DIM), jnp.float32),
            pltpu.VMEM((_BPW,), jnp.float32),
            pltpu.SemaphoreType.DMA,
            pltpu.SemaphoreType.DMA,
        ],
    )
    return f(xf, W_w, W_c)


def kernel(X, W_w, W_c):
    xf = jnp.reshape(X.astype(jnp.int32), (BATCH * 2,))
    out = _run(xf, W_w, W_c)
    return jnp.reshape(out, (BATCH, 1))


# speedup vs baseline: 3.6649x; 3.6649x over previous
import jax
import jax.numpy as jnp
from jax import lax
from jax.experimental import pallas as pl
from jax.experimental.pallas import tpu as pltpu
from jax.experimental.pallas import tpu_sc as plsc

INPUT_DIM = 1000000
PROJ_DIM = 32
BATCH = 16384

_info = plsc.get_sparse_core_info()
_NC, _NS, _L = _info.num_cores, _info.num_subcores, _info.num_lanes
_NW = _NC * _NS
_BPW = BATCH // _NW
_HB = _BPW // 2

_CHUNK = 8192
_NCHUNK = -(-INPUT_DIM // _CHUNK)
_QUART = _CHUNK // 4
_QROWS = _NCHUNK * _QUART


def _relayout_body(wwt_ref, wct_ref, pw_ref, pc_ref):
    for src, dst in ((wwt_ref, pw_ref), (wct_ref, pc_ref)):
        x = src[...]
        u = jnp.concatenate(
            [x[:, k * _QUART:(k + 1) * _QUART] for k in range(4)], axis=0)
        dst[...] = jnp.transpose(u)


def _relayout(wwt, wct):
    spec_in = pl.BlockSpec((PROJ_DIM, _CHUNK), lambda c: (0, c))
    spec_out = pl.BlockSpec((_QUART, 4 * PROJ_DIM), lambda c: (c, 0))
    oshape = jax.ShapeDtypeStruct((_QROWS, 4 * PROJ_DIM), jnp.float32)
    return pl.pallas_call(
        _relayout_body,
        grid=(_NCHUNK,),
        in_specs=[spec_in, spec_in],
        out_specs=[spec_out, spec_out],
        out_shape=[oshape, oshape],
    )(wwt, wct)


def _sc_body(xf_hbm, pw_hbm, pc_hbm, out_hbm,
             xf_v, idxq0_v, idxq1_v, ph0_v, ph1_v,
             rows_w, rows_c, out_v, sem0, sem1):
    wid = lax.axis_index("s") * _NC + lax.axis_index("c")
    base = wid * _BPW

    pltpu.sync_copy(xf_hbm.at[pl.ds(base * 2, _BPW * 2)], xf_v)

    lane_iota = lax.iota(jnp.int32, _L)

    for k in range(_BPW // _L):
        pos = lane_iota * 2 + (k * 2 * _L)
        r0 = plsc.load_gather(xf_v, [pos])
        r1 = plsc.load_gather(xf_v, [pos + 1])
        sl = pl.ds(k * _L, _L)
        idxq0_v[sl] = ((r0 >> 13) << 11) + (r0 & 2047)
        idxq1_v[sl] = ((r1 >> 13) << 11) + (r1 & 2047)
        ph0_v[sl] = ((r0 >> 11) & 3) * PROJ_DIM
        ph1_v[sl] = ((r1 >> 11) & 3) * PROJ_DIM

    def half(h, _):
        hb = h * _HB
        cp0 = pltpu.async_copy(
            pw_hbm.at[idxq0_v.at[pl.ds(hb, _HB)]], rows_w, sem0)
        cp1 = pltpu.async_copy(
            pc_hbm.at[idxq1_v.at[pl.ds(hb, _HB)]], rows_c, sem1)
        cp0.wait()
        cp1.wait()

        def block_body(b, _):
            row_idx = lane_iota + b * _L
            sl = pl.ds(hb + b * _L, _L)
            p0 = ph0_v[sl]
            p1 = ph1_v[sl]
            acc = jnp.zeros((_L,), jnp.float32)
            for j in range(PROJ_DIM):
                gw = plsc.load_gather(rows_w, [row_idx, p0 + j])
                gc = plsc.load_gather(rows_c, [row_idx, p1 + j])
                acc = acc + gw * gc
            out_v[sl] = 1.0 / (1.0 + jnp.exp(-acc))
            return 0

        lax.fori_loop(0, _HB // _L, block_body, 0)
        return 0

    lax.fori_loop(0, 2, half, 0)

    pltpu.sync_copy(out_v, out_hbm.at[pl.ds(base, _BPW)])


def _sc_gather(xf, pw, pc):
    mesh = plsc.VectorSubcoreMesh(core_axis_name="c", subcore_axis_name="s")
    f = pl.kernel(
        _sc_body,
        mesh=mesh,
        compiler_params=pltpu.CompilerParams(
            use_tc_tiling_on_sc=False, needs_layout_passes=False),
        out_type=jax.ShapeDtypeStruct((BATCH,), jnp.float32),
        scratch_types=[
            pltpu.VMEM((2 * _BPW,), jnp.int32),
            pltpu.VMEM((_BPW,), jnp.int32),
            pltpu.VMEM((_BPW,), jnp.int32),
            pltpu.VMEM((_BPW,), jnp.int32),
            pltpu.VMEM((_BPW,), jnp.int32),
            pltpu.VMEM((_HB, 4 * PROJ_DIM), jnp.float32),
            pltpu.VMEM((_HB, 4 * PROJ_DIM), jnp.float32),
            pltpu.VMEM((_BPW,), jnp.float32),
            pltpu.SemaphoreType.DMA,
            pltpu.SemaphoreType.DMA,
        ],
    )
    return f(xf, pw, pc)


@jax.jit
def _run(X, W_w, W_c):
    xf = jnp.reshape(X.astype(jnp.int32), (BATCH * 2,))
    pw, pc = _relayout(jnp.transpose(W_w), jnp.transpose(W_c))
    out = _sc_gather(xf, pw, pc)
    return jnp.reshape(out, (BATCH, 1))


def kernel(X, W_w, W_c):
    return _run(X, W_w, W_c)


# speedup vs baseline: 3.9358x; 1.0739x over previous
import jax
import jax.numpy as jnp
from jax import lax
from jax.experimental import pallas as pl
from jax.experimental.pallas import tpu as pltpu
from jax.experimental.pallas import tpu_sc as plsc

INPUT_DIM = 1000000
PROJ_DIM = 32
BATCH = 16384

_info = plsc.get_sparse_core_info()
_NC, _NS, _L = _info.num_cores, _info.num_subcores, _info.num_lanes
_NW = _NC * _NS
_BPW = BATCH // _NW

_CHUNK = 8192
_NCHUNK = -(-INPUT_DIM // _CHUNK)
_QUART = _CHUNK // 4
_QROWS = _NCHUNK * _QUART
_PROWS = 4 * _QROWS


def _relayout_body(wwt_ref, wct_ref, pw_ref, pc_ref):
    for src, dst in ((wwt_ref, pw_ref), (wct_ref, pc_ref)):
        x = src[...]
        u = jnp.concatenate(
            [x[:, k * _QUART:(k + 1) * _QUART] for k in range(4)], axis=0)
        dst[...] = jnp.transpose(u)


def _relayout(wwt, wct):
    spec_in = pl.BlockSpec((PROJ_DIM, _CHUNK), lambda c: (0, c))
    spec_out = pl.BlockSpec((_QUART, 4 * PROJ_DIM), lambda c: (c, 0))
    oshape = jax.ShapeDtypeStruct((_QROWS, 4 * PROJ_DIM), jnp.float32)
    return pl.pallas_call(
        _relayout_body,
        grid=(_NCHUNK,),
        in_specs=[spec_in, spec_in],
        out_specs=[spec_out, spec_out],
        out_shape=[oshape, oshape],
    )(wwt, wct)


def _packed_idx(r):
    return ((r >> 13) << 13) + ((r & 2047) << 2) + ((r >> 11) & 3)


def _sc_body(x2_hbm, pw_hbm, pc_hbm, out_hbm,
             x_v, idx0_v, idx1_v, rows_w, rows_c, out_v, sem0, sem1):
    wid = lax.axis_index("s") * _NC + lax.axis_index("c")
    base = wid * _BPW

    pltpu.sync_copy(x2_hbm.at[pl.ds(wid * 4, 4), :], x_v)

    for k in range(_BPW // _L):
        t, g = divmod(k * _L, 128)
        r0 = x_v[t, pl.ds(g, _L)]
        r1 = x_v[t, pl.ds(128 + g, _L)]
        idx0_v[pl.ds(k * _L, _L)] = _packed_idx(r0)
        idx1_v[pl.ds(k * _L, _L)] = _packed_idx(r1)

    cp0 = pltpu.async_copy(pw_hbm.at[idx0_v], rows_w, sem0)
    cp1 = pltpu.async_copy(pc_hbm.at[idx1_v], rows_c, sem1)
    cp0.wait()
    cp1.wait()

    lane_iota = lax.iota(jnp.int32, _L)

    def block_body(b, _):
        row_idx = lane_iota + b * _L
        acc = jnp.zeros((_L,), jnp.float32)
        for j in range(PROJ_DIM):
            col = jnp.full((_L,), j, jnp.int32)
            gw = plsc.load_gather(rows_w, [row_idx, col])
            gc = plsc.load_gather(rows_c, [row_idx, col])
            acc = acc + gw * gc
        out_v[pl.ds(b * _L, _L)] = 1.0 / (1.0 + jnp.exp(-acc))
        return 0

    lax.fori_loop(0, _BPW // _L, block_body, 0)

    pltpu.sync_copy(out_v, out_hbm.at[pl.ds(base, _BPW)])


def _sc_gather(x2, pw, pc):
    mesh = plsc.VectorSubcoreMesh(core_axis_name="c", subcore_axis_name="s")
    f = pl.kernel(
        _sc_body,
        mesh=mesh,
        compiler_params=pltpu.CompilerParams(
            use_tc_tiling_on_sc=False, needs_layout_passes=False),
        out_type=jax.ShapeDtypeStruct((BATCH,), jnp.float32),
        scratch_types=[
            pltpu.VMEM((4, 2 * 128), jnp.int32),
            pltpu.VMEM((_BPW,), jnp.int32),
            pltpu.VMEM((_BPW,), jnp.int32),
            pltpu.VMEM((_BPW, PROJ_DIM), jnp.float32),
            pltpu.VMEM((_BPW, PROJ_DIM), jnp.float32),
            pltpu.VMEM((_BPW,), jnp.float32),
            pltpu.SemaphoreType.DMA,
            pltpu.SemaphoreType.DMA,
        ],
    )
    return f(x2, pw, pc)


@jax.jit
def _run(X, W_w, W_c):
    xt = jnp.reshape(jnp.transpose(X.astype(jnp.int32)), (2, 128, 128))
    x2 = jnp.reshape(jnp.transpose(xt, (1, 0, 2)), (128, 2 * 128))
    pw, pc = _relayout(jnp.transpose(W_w), jnp.transpose(W_c))
    out = _sc_gather(x2,
                     jnp.reshape(pw, (_PROWS, PROJ_DIM)),
                     jnp.reshape(pc, (_PROWS, PROJ_DIM)))
    return jnp.reshape(out, (BATCH, 1))


def kernel(X, W_w, W_c):
    return _run(X, W_w, W_c)


# speedup vs baseline: 4.5335x; 1.1519x over previous
import jax
import jax.numpy as jnp
from jax import lax
from jax.experimental import pallas as pl
from jax.experimental.pallas import tpu as pltpu
from jax.experimental.pallas import tpu_sc as plsc

INPUT_DIM = 1000000
PROJ_DIM = 32
BATCH = 16384

_info = plsc.get_sparse_core_info()
_NC, _NS, _L = _info.num_cores, _info.num_subcores, _info.num_lanes
_NW = _NC * _NS
_BPW = BATCH // _NW

_CHUNK = 16384
_NCHUNK = -(-INPUT_DIM // _CHUNK)
_QUART = _CHUNK // 4
_QROWS = _NCHUNK * _QUART
_PROWS = 4 * _QROWS


def _relayout_body(wwt_ref, wct_ref, pw_ref, pc_ref):
    for src, dst in ((wwt_ref, pw_ref), (wct_ref, pc_ref)):
        x = src[...]
        u = jnp.concatenate(
            [x[:, k * _QUART:(k + 1) * _QUART] for k in range(4)], axis=0)
        dst[...] = jnp.transpose(u)


def _relayout(wwt, wct):
    spec_in = pl.BlockSpec((PROJ_DIM, _CHUNK), lambda c: (0, c))
    spec_out = pl.BlockSpec((_QUART, 4 * PROJ_DIM), lambda c: (c, 0))
    oshape = jax.ShapeDtypeStruct((_QROWS, 4 * PROJ_DIM), jnp.float32)
    return pl.pallas_call(
        _relayout_body,
        grid=(_NCHUNK,),
        in_specs=[spec_in, spec_in],
        out_specs=[spec_out, spec_out],
        out_shape=[oshape, oshape],
    )(wwt, wct)


_LOG_CHUNK = _CHUNK.bit_length() - 1
_LOG_QUART = _QUART.bit_length() - 1


def _packed_idx(r):
    return (((r >> _LOG_CHUNK) << _LOG_CHUNK)
            + ((r & (_QUART - 1)) << 2) + ((r >> _LOG_QUART) & 3))


def _sc_body(x2_hbm, pw_hbm, pc_hbm, out_hbm,
             x_v, idx0_v, idx1_v, rows_w, rows_c, out_v, sem0, sem1):
    wid = lax.axis_index("s") * _NC + lax.axis_index("c")
    base = wid * _BPW

    pltpu.sync_copy(x2_hbm.at[pl.ds(wid * 4, 4), :], x_v)

    for k in range(_BPW // _L):
        t, g = divmod(k * _L, 128)
        r0 = x_v[t, pl.ds(g, _L)]
        r1 = x_v[t, pl.ds(128 + g, _L)]
        idx0_v[pl.ds(k * _L, _L)] = _packed_idx(r0)
        idx1_v[pl.ds(k * _L, _L)] = _packed_idx(r1)

    cp0 = pltpu.async_copy(pw_hbm.at[idx0_v], rows_w, sem0)
    cp1 = pltpu.async_copy(pc_hbm.at[idx1_v], rows_c, sem1)
    cp0.wait()
    cp1.wait()

    lane_iota = lax.iota(jnp.int32, _L)

    def block_body(b, _):
        row_idx = lane_iota + b * _L
        acc = jnp.zeros((_L,), jnp.float32)
        for j in range(PROJ_DIM):
            col = jnp.full((_L,), j, jnp.int32)
            gw = plsc.load_gather(rows_w, [row_idx, col])
            gc = plsc.load_gather(rows_c, [row_idx, col])
            acc = acc + gw * gc
        out_v[pl.ds(b * _L, _L)] = 1.0 / (1.0 + jnp.exp(-acc))
        return 0

    lax.fori_loop(0, _BPW // _L, block_body, 0)

    pltpu.sync_copy(out_v, out_hbm.at[pl.ds(base, _BPW)])


def _sc_gather(x2, pw, pc):
    mesh = plsc.VectorSubcoreMesh(core_axis_name="c", subcore_axis_name="s")
    f = pl.kernel(
        _sc_body,
        mesh=mesh,
        compiler_params=pltpu.CompilerParams(
            use_tc_tiling_on_sc=False, needs_layout_passes=False),
        out_type=jax.ShapeDtypeStruct((BATCH,), jnp.float32),
        scratch_types=[
            pltpu.VMEM((4, 2 * 128), jnp.int32),
            pltpu.VMEM((_BPW,), jnp.int32),
            pltpu.VMEM((_BPW,), jnp.int32),
            pltpu.VMEM((_BPW, PROJ_DIM), jnp.float32),
            pltpu.VMEM((_BPW, PROJ_DIM), jnp.float32),
            pltpu.VMEM((_BPW,), jnp.float32),
            pltpu.SemaphoreType.DMA,
            pltpu.SemaphoreType.DMA,
        ],
    )
    return f(x2, pw, pc)


@jax.jit
def _run(X, W_w, W_c):
    xt = jnp.reshape(jnp.transpose(X.astype(jnp.int32)), (2, 128, 128))
    x2 = jnp.reshape(jnp.transpose(xt, (1, 0, 2)), (128, 2 * 128))
    pw, pc = _relayout(jnp.transpose(W_w), jnp.transpose(W_c))
    out = _sc_gather(x2,
                     jnp.reshape(pw, (_PROWS, PROJ_DIM)),
                     jnp.reshape(pc, (_PROWS, PROJ_DIM)))
    return jnp.reshape(out, (BATCH, 1))


def kernel(X, W_w, W_c):
    return _run(X, W_w, W_c)


# speedup vs baseline: 4.6397x; 1.0234x over previous
import jax
import jax.numpy as jnp
from jax import lax
from jax.experimental import pallas as pl
from jax.experimental.pallas import tpu as pltpu
from jax.experimental.pallas import tpu_sc as plsc

INPUT_DIM = 1000000
PROJ_DIM = 32
BATCH = 16384

_info = plsc.get_sparse_core_info()
_NC, _NS, _L = _info.num_cores, _info.num_subcores, _info.num_lanes
_NW = _NC * _NS
_BPW = BATCH // _NW

_CHUNK = 32768
_NCHUNK = -(-INPUT_DIM // _CHUNK)
_QUART = _CHUNK // 4
_QROWS = _NCHUNK * _QUART
_PROWS = 4 * _QROWS


def _relayout_body(wwt_ref, wct_ref, pw_ref, pc_ref):
    for src, dst in ((wwt_ref, pw_ref), (wct_ref, pc_ref)):
        x = src[...]
        u = jnp.concatenate(
            [x[:, k * _QUART:(k + 1) * _QUART] for k in range(4)], axis=0)
        dst[...] = jnp.transpose(u)


def _relayout(wwt, wct):
    spec_in = pl.BlockSpec((PROJ_DIM, _CHUNK), lambda c: (0, c))
    spec_out = pl.BlockSpec((_QUART, 4 * PROJ_DIM), lambda c: (c, 0))
    oshape = jax.ShapeDtypeStruct((_QROWS, 4 * PROJ_DIM), jnp.float32)
    return pl.pallas_call(
        _relayout_body,
        grid=(_NCHUNK,),
        in_specs=[spec_in, spec_in],
        out_specs=[spec_out, spec_out],
        out_shape=[oshape, oshape],
    )(wwt, wct)


_LOG_CHUNK = _CHUNK.bit_length() - 1
_LOG_QUART = _QUART.bit_length() - 1


def _packed_idx(r):
    return (((r >> _LOG_CHUNK) << _LOG_CHUNK)
            + ((r & (_QUART - 1)) << 2) + ((r >> _LOG_QUART) & 3))


def _sc_body(x2_hbm, pw_hbm, pc_hbm, out_hbm,
             x_v, idx0_v, idx1_v, rows_w, rows_c, out_v, sem0, sem1):
    wid = lax.axis_index("s") * _NC + lax.axis_index("c")
    base = wid * _BPW

    pltpu.sync_copy(x2_hbm.at[pl.ds(wid * 4, 4), :], x_v)

    for k in range(_BPW // _L):
        t, g = divmod(k * _L, 128)
        r0 = x_v[t, pl.ds(g, _L)]
        r1 = x_v[t, pl.ds(128 + g, _L)]
        idx0_v[pl.ds(k * _L, _L)] = _packed_idx(r0)
        idx1_v[pl.ds(k * _L, _L)] = _packed_idx(r1)

    cp0 = pltpu.async_copy(pw_hbm.at[idx0_v], rows_w, sem0)
    cp1 = pltpu.async_copy(pc_hbm.at[idx1_v], rows_c, sem1)
    cp0.wait()
    cp1.wait()

    lane_iota = lax.iota(jnp.int32, _L)

    def block_body(b, _):
        row_idx = lane_iota + b * _L
        acc = jnp.zeros((_L,), jnp.float32)
        for j in range(PROJ_DIM):
            col = jnp.full((_L,), j, jnp.int32)
            gw = plsc.load_gather(rows_w, [row_idx, col])
            gc = plsc.load_gather(rows_c, [row_idx, col])
            acc = acc + gw * gc
        out_v[pl.ds(b * _L, _L)] = 1.0 / (1.0 + jnp.exp(-acc))
        return 0

    lax.fori_loop(0, _BPW // _L, block_body, 0)

    pltpu.sync_copy(out_v, out_hbm.at[pl.ds(base, _BPW)])


def _sc_gather(x2, pw, pc):
    mesh = plsc.VectorSubcoreMesh(core_axis_name="c", subcore_axis_name="s")
    f = pl.kernel(
        _sc_body,
        mesh=mesh,
        compiler_params=pltpu.CompilerParams(
            use_tc_tiling_on_sc=False, needs_layout_passes=False),
        out_type=jax.ShapeDtypeStruct((BATCH,), jnp.float32),
        scratch_types=[
            pltpu.VMEM((4, 2 * 128), jnp.int32),
            pltpu.VMEM((_BPW,), jnp.int32),
            pltpu.VMEM((_BPW,), jnp.int32),
            pltpu.VMEM((_BPW, PROJ_DIM), jnp.float32),
            pltpu.VMEM((_BPW, PROJ_DIM), jnp.float32),
            pltpu.VMEM((_BPW,), jnp.float32),
            pltpu.SemaphoreType.DMA,
            pltpu.SemaphoreType.DMA,
        ],
    )
    return f(x2, pw, pc)


@jax.jit
def _run(X, W_w, W_c):
    xt = jnp.reshape(jnp.transpose(X.astype(jnp.int32)), (2, 128, 128))
    x2 = jnp.reshape(jnp.transpose(xt, (1, 0, 2)), (128, 2 * 128))
    pw, pc = _relayout(jnp.transpose(W_w), jnp.transpose(W_c))
    out = _sc_gather(x2,
                     jnp.reshape(pw, (_PROWS, PROJ_DIM)),
                     jnp.reshape(pc, (_PROWS, PROJ_DIM)))
    return jnp.reshape(out, (BATCH, 1))


def kernel(X, W_w, W_c):
    return _run(X, W_w, W_c)
